# Initial kernel scaffold; baseline (speedup 1.0000x reference)
import jax
import jax.numpy as jnp
from jax.experimental import pallas as pl


def kernel(node_feats, params, edge_index, graph_ids):
    raise NotImplementedError("write your pallas kernel here")


# speedup vs baseline: 9.2040x; 9.2040x over previous
import functools

import jax
import jax.numpy as jnp
from jax import lax
from jax.experimental import pallas as pl
from jax.experimental.pallas import tpu as pltpu
from jax.experimental.pallas import tpu_sc as plsc

_N = 10000
_E = 320000
_D_IN = 128
_D_H = 128
_D_OUT = 64
_B = 256
_T = 5
_NG = 4
_CH = 128
_EPS = 1e-5

_NPAD = 10240
_K = 80
_ECHUNKS = _E // _K
_W = 32
_CPW = _ECHUNKS // _W
_NBUF = 4
_SUP = 16
_NSUP = _CPW // _SUP
_CPT = _ECHUNKS // 16

_f32 = jnp.float32


@functools.lru_cache(maxsize=None)
def _sc_mesh():
    return plsc.VectorSubcoreMesh(core_axis_name="c", subcore_axis_name="s")


def _zero_vmem(ref, nrows, ncols):
    z = jnp.zeros((16,), _f32)

    def body(r, _):
        for j in range(ncols // 16):
            ref[r, pl.ds(j * 16, 16)] = z
        return 0

    lax.fori_loop(0, nrows, body, 0)


@functools.lru_cache(maxsize=None)
def _make_deg_kernel():
    return functools.partial(
        pl.kernel,
        mesh=_sc_mesh(),
        out_type=jax.ShapeDtypeStruct((2, _NPAD), _f32),
        scratch_types=[
            pltpu.VMEM((_CPT, _K), jnp.int32),
            pltpu.VMEM((1, _K), _f32),
            pltpu.VMEM((1, 640), _f32),
            pltpu.VMEM_SHARED((_NPAD,), _f32),
        ],
    )(_deg_body)


def _deg_body(ei_hbm, out_hbm, eidx_v, ones_v, zb_v, acc_sh):
    c = lax.axis_index("c")
    s = lax.axis_index("s")
    one = jnp.ones((16,), _f32)
    for j in range(_K // 16):
        ones_v[0, pl.ds(j * 16, 16)] = one
    _zero_vmem(zb_v, 1, 640)
    pltpu.sync_copy(zb_v.at[0], acc_sh.at[pl.ds(s * 640, 640)])
    plsc.subcore_barrier()
    pltpu.sync_copy(ei_hbm.at[c, s], eidx_v)

    def body(ci, _):
        pltpu.sync_copy(ones_v.at[0], acc_sh.at[eidx_v.at[ci]], add=True)
        return 0

    lax.fori_loop(0, _CPT, body, 0)
    plsc.subcore_barrier()
    pltpu.sync_copy(acc_sh.at[pl.ds(s * 640, 640)], out_hbm.at[c, pl.ds(s * 640, 640)])


@functools.lru_cache(maxsize=None)
def _make_agg_kernel(d):
    @functools.partial(
        pl.kernel,
        mesh=_sc_mesh(),
        out_type=jax.ShapeDtypeStruct((2, _NPAD, d), _f32),
        scratch_types=[
            pltpu.VMEM((_SUP, _K), jnp.int32),
            pltpu.VMEM((_SUP, _K), jnp.int32),
            pltpu.VMEM((_NBUF, _K, d), _f32),
            pltpu.VMEM_SHARED((_NPAD, d), _f32),
        ] + [pltpu.SemaphoreType.DMA] * _NBUF,
        compiler_params=pltpu.CompilerParams(use_tc_tiling_on_sc=False),
    )
    def _agg(h_hbm, ei_hbm, out_hbm, si16, di16, rb_v, acc_sh, *sems):
        c = lax.axis_index("c")
        s = lax.axis_index("s")
        w = s * 2 + c
        _zero_vmem(rb_v.at[0], _K, d)
        for j in range(640 // _K):
            pltpu.sync_copy(rb_v.at[0], acc_sh.at[pl.ds(s * 640 + j * _K, _K)])
        plsc.subcore_barrier()

        def fire_drain(base):
            descs = [
                pltpu.async_copy(h_hbm.at[si16.at[base + b]],
                                 rb_v.at[b], sems[b])
                for b in range(_NBUF)
            ]
            for b in range(_NBUF):
                descs[b].wait()
                pltpu.sync_copy(rb_v.at[b], acc_sh.at[di16.at[base + b]],
                                add=True)

        def super_body(g, _):
            pltpu.sync_copy(ei_hbm.at[0, w, pl.ds(g * _SUP, _SUP)], si16)
            pltpu.sync_copy(ei_hbm.at[1, w, pl.ds(g * _SUP, _SUP)], di16)
            for q in range(_SUP // _NBUF):
                fire_drain(q * _NBUF)
            return 0

        lax.fori_loop(0, _NSUP, super_body, 0)
        tail = _CPW - _NSUP * _SUP
        pltpu.sync_copy(ei_hbm.at[0, w, pl.ds(_NSUP * _SUP, tail)],
                        si16.at[pl.ds(0, tail)])
        pltpu.sync_copy(ei_hbm.at[1, w, pl.ds(_NSUP * _SUP, tail)],
                        di16.at[pl.ds(0, tail)])
        for q in range(tail // _NBUF):
            fire_drain(q * _NBUF)
        for r in range((tail // _NBUF) * _NBUF, tail):
            pltpu.async_copy(h_hbm.at[si16.at[r]], rb_v.at[0],
                             sems[0]).wait()
            pltpu.sync_copy(rb_v.at[0], acc_sh.at[di16.at[r]], add=True)
        plsc.subcore_barrier()
        pltpu.sync_copy(acc_sh.at[pl.ds(s * 640, 640)],
                        out_hbm.at[c, pl.ds(s * 640, 640)])

    return _agg


def _tca_body(x_ref, deg_ref, w1_ref, rw1_ref, rb1_ref,
              h1_ref, res1_ref, so_ref, si_ref):
    x = x_ref[...]
    deg = deg_ref[...]
    so = lax.rsqrt(jnp.maximum(deg[0], 1.0))
    si = lax.rsqrt(jnp.maximum(deg[1], 1.0))
    so_ref[...] = so
    si_ref[...] = si
    xw = jnp.dot(x, w1_ref[...], preferred_element_type=_f32, precision=lax.Precision.HIGHEST)
    h1_ref[...] = xw * so[:_N][:, None]
    res1_ref[...] = jax.nn.relu(
        jnp.dot(x, rw1_ref[...], preferred_element_type=_f32, precision=lax.Precision.HIGHEST) + rb1_ref[...])


def _tc_a(x, deg, w1, rw1, rb1):
    return pl.pallas_call(
        _tca_body,
        out_shape=[
            jax.ShapeDtypeStruct((_N, _D_H), _f32),
            jax.ShapeDtypeStruct((_N, _D_H), _f32),
            jax.ShapeDtypeStruct((_NPAD,), _f32),
            jax.ShapeDtypeStruct((_NPAD,), _f32),
        ],
    )(x, deg, w1, rw1, rb1)


def _tcb_body(p_ref, res1_ref, si_ref, so_ref, b1_ref, g1_ref, bb1_ref,
              w2_ref, rw2_ref, rb2_ref, h2p_ref, res2_ref):
    agg = p_ref[0, :_N, :] + p_ref[1, :_N, :]
    si = si_ref[...][:_N]
    so = so_ref[...][:_N]
    t = jax.nn.relu(agg * si[:, None] + b1_ref[...]) + res1_ref[...]
    mu = jnp.mean(t, axis=0)
    ctr = t - mu
    var = jnp.mean(ctr * ctr, axis=0)
    a = g1_ref[...] * lax.rsqrt(var + _EPS)
    cshift = bb1_ref[...] - mu * a
    w2 = w2_ref[...]
    rw2 = rw2_ref[...]
    tw2 = jnp.dot(t, a[:, None] * w2, preferred_element_type=_f32, precision=lax.Precision.HIGHEST)
    cw2 = jnp.dot(cshift[None, :], w2, preferred_element_type=_f32, precision=lax.Precision.HIGHEST)
    h2p_ref[...] = so[:, None] * (tw2 + cw2)
    trw = jnp.dot(t, a[:, None] * rw2, preferred_element_type=_f32, precision=lax.Precision.HIGHEST)
    crw = jnp.dot(cshift[None, :], rw2, preferred_element_type=_f32, precision=lax.Precision.HIGHEST)
    res2_ref[...] = jax.nn.relu(trw + crw + rb2_ref[...])


def _tc_b(parts, res1, si, so, b1, g1, bb1, w2, rw2, rb2):
    return pl.pallas_call(
        _tcb_body,
        out_shape=[
            jax.ShapeDtypeStruct((_N, _D_OUT), _f32),
            jax.ShapeDtypeStruct((_N, _D_OUT), _f32),
        ],
    )(parts, res1, si, so, b1, g1, bb1, w2, rw2, rb2)


def _bn(z, g, b):
    mu = jnp.mean(z, axis=0)
    ctr = z - mu
    var = jnp.mean(ctr * ctr, axis=0)
    return ctr * lax.rsqrt(var + _EPS) * g + b


def _sigmoid(x):
    return 1.0 / (1.0 + jnp.exp(-x))


def _tcc0_body(p_ref, res2_ref, si_ref, b2_ref, g2_ref, bb2_ref,
               t2_ref, ab_ref):
    agg = p_ref[0, :_N, :] + p_ref[1, :_N, :]
    si = si_ref[...][:_N]
    t = jax.nn.relu(agg * si[:, None] + b2_ref[...]) + res2_ref[...]
    mu = jnp.mean(t, axis=0)
    ctr = t - mu
    var = jnp.mean(ctr * ctr, axis=0)
    a2 = g2_ref[...] * lax.rsqrt(var + _EPS)
    c2 = bb2_ref[...] - mu * a2
    t2_ref[...] = t
    ab_ref[0, :] = a2
    ab_ref[1, :] = c2


def _tc_c0(parts, res2, si, b2, g2, bb2):
    return pl.pallas_call(
        _tcc0_body,
        out_shape=[
            jax.ShapeDtypeStruct((_N, _D_OUT), _f32),
            jax.ShapeDtypeStruct((2, _D_OUT), _f32),
        ],
    )(parts, res2, si, b2, g2, bb2)


_NB = 1000
_NSTEPS = _N // _NB


def _tcc1_body(t2_ref, ab_ref, aw_ref, awb_ref, gid_ref,
               s5_ref, sh_ref, cn_ref):
    t2b = t2_ref[...]
    h2b = t2b * ab_ref[0, :] + ab_ref[1, :]
    wts = _sigmoid(jnp.dot(h2b, aw_ref[...].T, preferred_element_type=_f32,
                           precision=lax.Precision.HIGHEST) + awb_ref[...])
    gidb = gid_ref[0, 0, :]
    row = lax.broadcasted_iota(jnp.int32, (_B, _NB), 0)
    m = jnp.where(row == gidb[None, :], 1.0, 0.0)
    upd = jnp.stack(
        [jnp.dot(m, h2b * wts[:, t][:, None], preferred_element_type=_f32,
                 precision=lax.Precision.HIGHEST) for t in range(_T)], axis=0)
    s5_ref[...] = upd[None]
    sh_ref[...] = jnp.dot(m, h2b, preferred_element_type=_f32,
                          precision=lax.Precision.HIGHEST)[None]
    cn_ref[...] = jnp.sum(m, axis=1)[None, None, :]


def _tc_c1(t2, ab, aw, awb, gid3):
    return pl.pallas_call(
        _tcc1_body,
        grid=(_NSTEPS,),
        in_specs=[
            pl.BlockSpec((_NB, _D_OUT), lambda i: (i, 0)),
            pl.BlockSpec((2, _D_OUT), lambda i: (0, 0)),
            pl.BlockSpec((_T, _D_OUT), lambda i: (0, 0)),
            pl.BlockSpec((_T,), lambda i: (0,)),
            pl.BlockSpec((1, 1, _NB), lambda i: (i, 0, 0)),
        ],
        out_specs=[
            pl.BlockSpec((1, _T, _B, _D_OUT), lambda i: (i, 0, 0, 0)),
            pl.BlockSpec((1, _B, _D_OUT), lambda i: (i, 0, 0)),
            pl.BlockSpec((1, 1, _B), lambda i: (i, 0, 0)),
        ],
        out_shape=[
            jax.ShapeDtypeStruct((_NSTEPS, _T, _B, _D_OUT), _f32),
            jax.ShapeDtypeStruct((_NSTEPS, _B, _D_OUT), _f32),
            jax.ShapeDtypeStruct((_NSTEPS, 1, _B), _f32),
        ],
    )(t2, ab, aw, awb, gid3)


def _tcc2_body(s5_ref, sh_ref, cn_ref, gw_ref, gb_ref,
               f1w_ref, f1b_ref, n1g_ref, n1b_ref,
               f2w_ref, f2b_ref, n2g_ref, n2b_ref,
               ow_ref, ob_ref, out_ref):
    counts = jnp.maximum(jnp.sum(cn_ref[...], axis=(0, 1)), 1.0)
    s5 = jnp.sum(s5_ref[...], axis=0)
    seg = [s5[t] for t in range(_T)]
    hg = jnp.sum(sh_ref[...], axis=0) / counts[:, None]
    combine1 = jnp.zeros((_B, _D_OUT), _f32)
    for i in range(_NG):
        logits = jnp.dot(hg, gw_ref[i], preferred_element_type=_f32,
                         precision=lax.Precision.HIGHEST) + gb_ref[i]
        lmax = jnp.max(logits, axis=-1, keepdims=True)
        e = jnp.exp(logits - lmax)
        gate = e / jnp.sum(e, axis=-1, keepdims=True)
        combine1 = combine1 + gate[:, 0:1] * seg[i] + gate[:, 1:2] * seg[_NG]
    combine2 = [seg[0], combine1, seg[1], seg[2], seg[3]]
    cols = []
    for i in range(_T):
        mfeat = combine2[i]
        z = jax.nn.relu(jnp.dot(mfeat, f1w_ref[i], preferred_element_type=_f32,
                                precision=lax.Precision.HIGHEST) + f1b_ref[i])
        z = _bn(z, n1g_ref[i], n1b_ref[i])
        z = jax.nn.relu(jnp.dot(z, f2w_ref[i], preferred_element_type=_f32,
                                precision=lax.Precision.HIGHEST) + f2b_ref[i])
        z = _bn(z, n2g_ref[i], n2b_ref[i])
        cols.append(jnp.dot(z, ow_ref[i][:, None], preferred_element_type=_f32,
                            precision=lax.Precision.HIGHEST) + ob_ref[i])
    out_ref[...] = jnp.concatenate(cols, axis=1)


def _tc_c2(s5, sh, cn, gw, gb, f1w, f1b, n1g, n1b, f2w, f2b, n2g, n2b, ow, ob):
    return pl.pallas_call(
        _tcc2_body,
        out_shape=jax.ShapeDtypeStruct((_B, _T), _f32),
    )(s5, sh, cn, gw, gb, f1w, f1b, n1g, n1b, f2w, f2b, n2g, n2b, ow, ob)


def kernel(node_feats, params, edge_index, graph_ids):
    p = params
    ei_deg = edge_index.reshape(2, 16, _CPT, _K)
    ei_agg = edge_index.reshape(2, _W, _CPW, _K)
    deg = _make_deg_kernel()(ei_deg)
    h1pre, res1, s_out, s_in = _tc_a(
        node_feats, deg, p['conv1_W'], p['res1_W'], p['res1_b'])
    parts1 = _make_agg_kernel(_D_H)(h1pre, ei_agg)
    h2pre, res2 = _tc_b(
        parts1, res1, s_in, s_out, p['conv1_b'], p['bnc1_g'], p['bnc1_b'],
        p['conv2_W'], p['res2_W'], p['res2_b'])
    parts2 = _make_agg_kernel(_D_OUT)(h2pre, ei_agg)
    t2, ab = _tc_c0(parts2, res2, s_in, p['conv2_b'], p['bnc2_g'], p['bnc2_b'])
    gid3 = graph_ids.reshape(_NSTEPS, 1, _NB)
    s5, sh, cn = _tc_c1(t2, ab, p['aw_W'], p['aw_b'], gid3)
    return _tc_c2(
        s5, sh, cn, p['gate_W'], p['gate_b'],
        p['fc1_W'], p['fc1_b'], p['bn1_g'], p['bn1_b'],
        p['fc2_W'], p['fc2_b'], p['bn2_g'], p['bn2_b'],
        p['out_W'], p['out_b'])


# speedup vs baseline: 9.4024x; 1.0216x over previous
import functools

import jax
import jax.numpy as jnp
from jax import lax
from jax.experimental import pallas as pl
from jax.experimental.pallas import tpu as pltpu
from jax.experimental.pallas import tpu_sc as plsc

_N = 10000
_E = 320000
_D_IN = 128
_D_H = 128
_D_OUT = 64
_B = 256
_T = 5
_NG = 4
_CH = 128
_EPS = 1e-5

_NPAD = 10240
_K = 80
_ECHUNKS = _E // _K
_W = 32
_CPW = _ECHUNKS // _W
_NBUF = 4
_SUP = 16
_NSUP = _CPW // _SUP
_CPT = _ECHUNKS // 16

_f32 = jnp.float32


@functools.lru_cache(maxsize=None)
def _sc_mesh():
    return plsc.VectorSubcoreMesh(core_axis_name="c", subcore_axis_name="s")


def _zero_vmem(ref, nrows, ncols):
    z = jnp.zeros((16,), _f32)

    def body(r, _):
        for j in range(ncols // 16):
            ref[r, pl.ds(j * 16, 16)] = z
        return 0

    lax.fori_loop(0, nrows, body, 0)


@functools.lru_cache(maxsize=None)
def _make_deg_kernel():
    return functools.partial(
        pl.kernel,
        mesh=_sc_mesh(),
        out_type=jax.ShapeDtypeStruct((2, _NPAD), _f32),
        scratch_types=[
            pltpu.VMEM((_CPT, _K), jnp.int32),
            pltpu.VMEM((1, _K), _f32),
            pltpu.VMEM((1, 640), _f32),
            pltpu.VMEM_SHARED((_NPAD,), _f32),
        ],
    )(_deg_body)


def _deg_body(ei_hbm, out_hbm, eidx_v, ones_v, zb_v, acc_sh):
    c = lax.axis_index("c")
    s = lax.axis_index("s")
    one = jnp.ones((16,), _f32)
    for j in range(_K // 16):
        ones_v[0, pl.ds(j * 16, 16)] = one
    _zero_vmem(zb_v, 1, 640)
    pltpu.sync_copy(zb_v.at[0], acc_sh.at[pl.ds(s * 640, 640)])
    plsc.subcore_barrier()
    pltpu.sync_copy(ei_hbm.at[c, s], eidx_v)

    def body(ci, _):
        pltpu.sync_copy(ones_v.at[0], acc_sh.at[eidx_v.at[ci]], add=True)
        return 0

    lax.fori_loop(0, _CPT, body, 0)
    plsc.subcore_barrier()
    pltpu.sync_copy(acc_sh.at[pl.ds(s * 640, 640)], out_hbm.at[c, pl.ds(s * 640, 640)])


@functools.lru_cache(maxsize=None)
def _make_agg_kernel(d):
    @functools.partial(
        pl.kernel,
        mesh=_sc_mesh(),
        out_type=jax.ShapeDtypeStruct((2, _NPAD, d), _f32),
        scratch_types=[
            pltpu.VMEM((_SUP, _K), jnp.int32),
            pltpu.VMEM((_SUP, _K), jnp.int32),
            pltpu.VMEM((_NBUF, _K, d), _f32),
            pltpu.VMEM_SHARED((_NPAD, d), _f32),
        ] + [pltpu.SemaphoreType.DMA] * _NBUF,
        compiler_params=pltpu.CompilerParams(use_tc_tiling_on_sc=False),
    )
    def _agg(h_hbm, ei_hbm, out_hbm, si16, di16, rb_v, acc_sh, *sems):
        c = lax.axis_index("c")
        s = lax.axis_index("s")
        w = s * 2 + c
        _zero_vmem(rb_v.at[0], _K, d)
        for j in range(640 // _K):
            pltpu.sync_copy(rb_v.at[0], acc_sh.at[pl.ds(s * 640 + j * _K, _K)])
        plsc.subcore_barrier()

        def fire_drain(base):
            descs = [
                pltpu.async_copy(h_hbm.at[si16.at[base + b]],
                                 rb_v.at[b], sems[b])
                for b in range(_NBUF)
            ]
            for b in range(_NBUF):
                descs[b].wait()
                pltpu.sync_copy(rb_v.at[b], acc_sh.at[di16.at[base + b]],
                                add=True)

        def super_body(g, _):
            pltpu.sync_copy(ei_hbm.at[0, w, pl.ds(g * _SUP, _SUP)], si16)
            pltpu.sync_copy(ei_hbm.at[1, w, pl.ds(g * _SUP, _SUP)], di16)
            for q in range(_SUP // _NBUF):
                fire_drain(q * _NBUF)
            return 0

        lax.fori_loop(0, _NSUP, super_body, 0)
        tail = _CPW - _NSUP * _SUP
        pltpu.sync_copy(ei_hbm.at[0, w, pl.ds(_NSUP * _SUP, tail)],
                        si16.at[pl.ds(0, tail)])
        pltpu.sync_copy(ei_hbm.at[1, w, pl.ds(_NSUP * _SUP, tail)],
                        di16.at[pl.ds(0, tail)])
        for q in range(tail // _NBUF):
            fire_drain(q * _NBUF)
        for r in range((tail // _NBUF) * _NBUF, tail):
            pltpu.async_copy(h_hbm.at[si16.at[r]], rb_v.at[0],
                             sems[0]).wait()
            pltpu.sync_copy(rb_v.at[0], acc_sh.at[di16.at[r]], add=True)
        plsc.subcore_barrier()
        pltpu.sync_copy(acc_sh.at[pl.ds(s * 640, 640)],
                        out_hbm.at[c, pl.ds(s * 640, 640)])

    return _agg


def _tca_body(x_ref, deg_ref, w1_ref, rw1_ref, rb1_ref,
              h1_ref, res1_ref, so_ref, si_ref):
    x = x_ref[...]
    deg = deg_ref[...]
    so = lax.rsqrt(jnp.maximum(deg[0], 1.0))
    si = lax.rsqrt(jnp.maximum(deg[1], 1.0))
    so_ref[...] = so
    si_ref[...] = si
    xw = jnp.dot(x, w1_ref[...], preferred_element_type=_f32, precision=lax.Precision.HIGHEST)
    h1_ref[...] = xw * so[:_N][:, None]
    res1_ref[...] = jax.nn.relu(
        jnp.dot(x, rw1_ref[...], preferred_element_type=_f32, precision=lax.Precision.HIGHEST) + rb1_ref[...])


def _tc_a(x, deg, w1, rw1, rb1):
    return pl.pallas_call(
        _tca_body,
        out_shape=[
            jax.ShapeDtypeStruct((_N, _D_H), _f32),
            jax.ShapeDtypeStruct((_N, _D_H), _f32),
            jax.ShapeDtypeStruct((_NPAD,), _f32),
            jax.ShapeDtypeStruct((_NPAD,), _f32),
        ],
    )(x, deg, w1, rw1, rb1)


def _tcb_body(p_ref, res1_ref, si_ref, so_ref, b1_ref, g1_ref, bb1_ref,
              w2_ref, rw2_ref, rb2_ref, h2p_ref, res2_ref):
    agg = p_ref[0, :_N, :] + p_ref[1, :_N, :]
    si = si_ref[...][:_N]
    so = so_ref[...][:_N]
    t = jax.nn.relu(agg * si[:, None] + b1_ref[...]) + res1_ref[...]
    mu = jnp.mean(t, axis=0)
    ctr = t - mu
    var = jnp.mean(ctr * ctr, axis=0)
    a = g1_ref[...] * lax.rsqrt(var + _EPS)
    cshift = bb1_ref[...] - mu * a
    w2 = w2_ref[...]
    rw2 = rw2_ref[...]
    tw2 = jnp.dot(t, a[:, None] * w2, preferred_element_type=_f32, precision=lax.Precision.HIGHEST)
    cw2 = jnp.dot(cshift[None, :], w2, preferred_element_type=_f32, precision=lax.Precision.HIGHEST)
    h2p_ref[...] = so[:, None] * (tw2 + cw2)
    trw = jnp.dot(t, a[:, None] * rw2, preferred_element_type=_f32, precision=lax.Precision.HIGHEST)
    crw = jnp.dot(cshift[None, :], rw2, preferred_element_type=_f32, precision=lax.Precision.HIGHEST)
    res2_ref[...] = jax.nn.relu(trw + crw + rb2_ref[...])


def _tc_b(parts, res1, si, so, b1, g1, bb1, w2, rw2, rb2):
    return pl.pallas_call(
        _tcb_body,
        out_shape=[
            jax.ShapeDtypeStruct((_N, _D_OUT), _f32),
            jax.ShapeDtypeStruct((_N, _D_OUT), _f32),
        ],
    )(parts, res1, si, so, b1, g1, bb1, w2, rw2, rb2)


def _bn(z, g, b):
    mu = jnp.mean(z, axis=0)
    ctr = z - mu
    var = jnp.mean(ctr * ctr, axis=0)
    return ctr * lax.rsqrt(var + _EPS) * g + b


def _sigmoid(x):
    return 1.0 / (1.0 + jnp.exp(-x))


def _tcc0_body(p_ref, res2_ref, si_ref, b2_ref, g2_ref, bb2_ref,
               t2_ref, ab_ref):
    agg = p_ref[0, :_N, :] + p_ref[1, :_N, :]
    si = si_ref[...][:_N]
    t = jax.nn.relu(agg * si[:, None] + b2_ref[...]) + res2_ref[...]
    mu = jnp.mean(t, axis=0)
    ctr = t - mu
    var = jnp.mean(ctr * ctr, axis=0)
    a2 = g2_ref[...] * lax.rsqrt(var + _EPS)
    c2 = bb2_ref[...] - mu * a2
    t2_ref[...] = t
    ab_ref[0, :] = a2
    ab_ref[1, :] = c2


def _tc_c0(parts, res2, si, b2, g2, bb2):
    return pl.pallas_call(
        _tcc0_body,
        out_shape=[
            jax.ShapeDtypeStruct((_N, _D_OUT), _f32),
            jax.ShapeDtypeStruct((2, _D_OUT), _f32),
        ],
    )(parts, res2, si, b2, g2, bb2)


_NB = 2000
_NSTEPS = _N // _NB


def _tcc1_body(t2_ref, ab_ref, aw_ref, awb_ref, gid_ref,
               gw_ref, gb_ref, f1w_ref, f1b_ref, n1g_ref, n1b_ref,
               f2w_ref, f2b_ref, n2g_ref, n2b_ref, ow_ref, ob_ref,
               out_ref, s5_acc, sh_acc, cn_acc):
    i = pl.program_id(0)
    t2b = t2_ref[...]
    h2b = t2b * ab_ref[0, :] + ab_ref[1, :]
    wts = _sigmoid(jnp.dot(h2b, aw_ref[...].T, preferred_element_type=_f32,
                           precision=lax.Precision.HIGHEST) + awb_ref[...])
    gidb = gid_ref[0, 0, :]
    row = lax.broadcasted_iota(jnp.int32, (_B, _NB), 0)
    m = jnp.where(row == gidb[None, :], 1.0, 0.0)
    upd = jnp.stack(
        [jnp.dot(m, h2b * wts[:, t][:, None], preferred_element_type=_f32,
                 precision=lax.Precision.HIGHEST) for t in range(_T)], axis=0)
    shu = jnp.dot(m, h2b, preferred_element_type=_f32,
                  precision=lax.Precision.HIGHEST)
    cnu = jnp.sum(m, axis=1)[None, :]

    @pl.when(i == 0)
    def _():
        s5_acc[...] = upd
        sh_acc[...] = shu
        cn_acc[...] = cnu

    @pl.when(i > 0)
    def _():
        s5_acc[...] += upd
        sh_acc[...] += shu
        cn_acc[...] += cnu

    @pl.when(i == _NSTEPS - 1)
    def _():
        counts = jnp.maximum(cn_acc[0, :], 1.0)
        seg = [s5_acc[t] for t in range(_T)]
        hg = sh_acc[...] / counts[:, None]
        combine1 = jnp.zeros((_B, _D_OUT), _f32)
        for g in range(_NG):
            logits = jnp.dot(hg, gw_ref[g], preferred_element_type=_f32,
                             precision=lax.Precision.HIGHEST) + gb_ref[g]
            lmax = jnp.max(logits, axis=-1, keepdims=True)
            e = jnp.exp(logits - lmax)
            gate = e / jnp.sum(e, axis=-1, keepdims=True)
            combine1 = (combine1 + gate[:, 0:1] * seg[g]
                        + gate[:, 1:2] * seg[_NG])
        combine2 = [seg[0], combine1, seg[1], seg[2], seg[3]]
        cols = []
        for t in range(_T):
            z = jax.nn.relu(
                jnp.dot(combine2[t], f1w_ref[t], preferred_element_type=_f32,
                        precision=lax.Precision.HIGHEST) + f1b_ref[t])
            z = _bn(z, n1g_ref[t], n1b_ref[t])
            z = jax.nn.relu(
                jnp.dot(z, f2w_ref[t], preferred_element_type=_f32,
                        precision=lax.Precision.HIGHEST) + f2b_ref[t])
            z = _bn(z, n2g_ref[t], n2b_ref[t])
            cols.append(
                jnp.dot(z, ow_ref[t][:, None], preferred_element_type=_f32,
                        precision=lax.Precision.HIGHEST) + ob_ref[t])
        out_ref[...] = jnp.concatenate(cols, axis=1)


def _tc_c1(t2, ab, aw, awb, gid3, gw, gb,
           f1w, f1b, n1g, n1b, f2w, f2b, n2g, n2b, ow, ob):
    full = lambda *shape: pl.BlockSpec(shape, lambda i: (0,) * len(shape))
    return pl.pallas_call(
        _tcc1_body,
        grid=(_NSTEPS,),
        in_specs=[
            pl.BlockSpec((_NB, _D_OUT), lambda i: (i, 0)),
            full(2, _D_OUT),
            full(_T, _D_OUT),
            full(_T),
            pl.BlockSpec((1, 1, _NB), lambda i: (i, 0, 0)),
            full(_T, _D_OUT, 2),
            full(_T, 2),
            full(_T, _D_OUT, _CH),
            full(_T, _CH),
            full(_T, _CH),
            full(_T, _CH),
            full(_T, _CH, _CH),
            full(_T, _CH),
            full(_T, _CH),
            full(_T, _CH),
            full(_T, _CH),
            full(_T),
        ],
        out_specs=pl.BlockSpec((_B, _T), lambda i: (0, 0)),
        out_shape=jax.ShapeDtypeStruct((_B, _T), _f32),
        scratch_shapes=[
            pltpu.VMEM((_T, _B, _D_OUT), _f32),
            pltpu.VMEM((_B, _D_OUT), _f32),
            pltpu.VMEM((1, _B), _f32),
        ],
    )(t2, ab, aw, awb, gid3, gw, gb,
      f1w, f1b, n1g, n1b, f2w, f2b, n2g, n2b, ow, ob)


def kernel(node_feats, params, edge_index, graph_ids):
    p = params
    ei_deg = edge_index.reshape(2, 16, _CPT, _K)
    ei_agg = edge_index.reshape(2, _W, _CPW, _K)
    deg = _make_deg_kernel()(ei_deg)
    h1pre, res1, s_out, s_in = _tc_a(
        node_feats, deg, p['conv1_W'], p['res1_W'], p['res1_b'])
    parts1 = _make_agg_kernel(_D_H)(h1pre, ei_agg)
    h2pre, res2 = _tc_b(
        parts1, res1, s_in, s_out, p['conv1_b'], p['bnc1_g'], p['bnc1_b'],
        p['conv2_W'], p['res2_W'], p['res2_b'])
    parts2 = _make_agg_kernel(_D_OUT)(h2pre, ei_agg)
    t2, ab = _tc_c0(parts2, res2, s_in, p['conv2_b'], p['bnc2_g'], p['bnc2_b'])
    gid3 = graph_ids.reshape(_NSTEPS, 1, _NB)
    return _tc_c1(
        t2, ab, p['aw_W'], p['aw_b'], gid3, p['gate_W'], p['gate_b'],
        p['fc1_W'], p['fc1_b'], p['bn1_g'], p['bn1_b'],
        p['fc2_W'], p['fc2_b'], p['bn2_g'], p['bn2_b'],
        p['out_W'], p['out_b'])
